# Initial kernel scaffold; baseline (speedup 1.0000x reference)
import jax
import jax.numpy as jnp
from jax.experimental import pallas as pl


def kernel(x, edge_index, W1, b1, W2, b2, Wlin, blin, Wskip, bskip):
    raise NotImplementedError("write your pallas kernel here")


# speedup vs baseline: 5.5122x; 5.5122x over previous
import functools

import jax
import jax.numpy as jnp
from jax import lax
from jax.experimental import pallas as pl
from jax.experimental.pallas import tpu as pltpu
from jax.experimental.pallas import tpu_sc as plsc

N = 10000
E = 160000
D = 256
H = 64
NT = 16
EPT = E // NT
CH = 80
NCH = EPT // CH
NRC = N // CH

_mesh = plsc.VectorSubcoreMesh(core_axis_name="c", subcore_axis_name="s")


@functools.partial(
    pl.kernel,
    mesh=_mesh,
    out_type=[jax.ShapeDtypeStruct((N,), jnp.float32),
              jax.ShapeDtypeStruct((N,), jnp.float32)],
    scratch_types=[
        pltpu.VMEM((NCH, CH), jnp.int32),
        pltpu.VMEM((128,), jnp.float32),
        pltpu.VMEM((N,), jnp.float32),
        pltpu.VMEM_SHARED((N,), jnp.float32),
    ],
)
def _deg_kernel(src_hbm, dst_hbm, ones_hbm, zeros_hbm, dout_hbm, din_hbm,
                idx_v, ones_v, stage_v, deg_sh):
    cid = lax.axis_index("c")
    sid = lax.axis_index("s")

    @pl.when(sid == 0)
    def _():
        pltpu.sync_copy(zeros_hbm, stage_v)
        pltpu.sync_copy(stage_v, deg_sh)

    pltpu.sync_copy(ones_hbm, ones_v)
    @pl.when(cid == 0)
    def _():
        pltpu.sync_copy(src_hbm.at[sid], idx_v)

    @pl.when(cid == 1)
    def _():
        pltpu.sync_copy(dst_hbm.at[sid], idx_v)

    plsc.subcore_barrier()

    def body(j, _):
        pltpu.sync_copy(ones_v.at[pl.ds(0, CH)], deg_sh.at[idx_v.at[j]],
                        add=True)
        return 0

    lax.fori_loop(0, NCH, body, 0)
    plsc.subcore_barrier()

    @pl.when((cid == 0) & (sid == 0))
    def _():
        pltpu.sync_copy(deg_sh, stage_v)
        pltpu.sync_copy(stage_v, dout_hbm)

    @pl.when((cid == 1) & (sid == 0))
    def _():
        pltpu.sync_copy(deg_sh, stage_v)
        pltpu.sync_copy(stage_v, din_hbm)


@functools.partial(
    pl.kernel,
    mesh=_mesh,
    out_type=[jax.ShapeDtypeStruct((N, H), jnp.float32)] * 4,
    scratch_types=[
        pltpu.VMEM((NCH, CH), jnp.int32),
        pltpu.VMEM((NCH, CH), jnp.int32),
        pltpu.VMEM((CH, H), jnp.float32),
        pltpu.VMEM((CH, H), jnp.float32),
        pltpu.VMEM_SHARED((N, H), jnp.float32),
        pltpu.SemaphoreType.DMA,
        pltpu.SemaphoreType.DMA,
    ],
    compiler_params=pltpu.CompilerParams(use_tc_tiling_on_sc=False),
)
def _mp_kernel(h0_hbm, h1_hbm, h2_hbm, h3_hbm, src_hbm, dst_hbm, zrow_hbm,
               o0_hbm, o1_hbm, o2_hbm, o3_hbm,
               src_v, dst_v, g0, g1, agg_sh, sem0, sem1):
    cid = lax.axis_index("c")
    sid = lax.axis_index("s")

    pltpu.sync_copy(src_hbm.at[sid], src_v)
    pltpu.sync_copy(dst_hbm.at[sid], dst_v)

    def run_block(h_hbm, o_hbm):
        pltpu.sync_copy(zrow_hbm, g0)
        for k in range(8):
            cidx = sid + NT * k

            @pl.when(cidx < NRC)
            def _():
                pltpu.sync_copy(g0, agg_sh.at[pl.ds(cidx * CH, CH)])

        plsc.subcore_barrier()

        pltpu.async_copy(h_hbm.at[src_v.at[0]], g0, sem0)
        pltpu.async_copy(h_hbm.at[src_v.at[1]], g1, sem1)

        def body(i, _):
            j0 = 2 * i
            pltpu.make_async_copy(h_hbm.at[pl.ds(0, CH)], g0, sem0).wait()
            pltpu.sync_copy(g0, agg_sh.at[dst_v.at[j0]], add=True)

            @pl.when(j0 + 2 < NCH)
            def _():
                pltpu.async_copy(h_hbm.at[src_v.at[j0 + 2]], g0, sem0)

            pltpu.make_async_copy(h_hbm.at[pl.ds(0, CH)], g1, sem1).wait()
            pltpu.sync_copy(g1, agg_sh.at[dst_v.at[j0 + 1]], add=True)

            @pl.when(j0 + 3 < NCH)
            def _():
                pltpu.async_copy(h_hbm.at[src_v.at[j0 + 3]], g1, sem1)

            return 0

        lax.fori_loop(0, NCH // 2, body, 0)
        pltpu.make_async_copy(h_hbm.at[pl.ds(0, CH)], g0, sem0).wait()
        pltpu.sync_copy(g0, agg_sh.at[dst_v.at[NCH - 1]], add=True)

        plsc.subcore_barrier()

        for k in range(8):
            cidx = sid + NT * k

            @pl.when(cidx < NRC)
            def _():
                pltpu.sync_copy(agg_sh.at[pl.ds(cidx * CH, CH)], g0)
                pltpu.sync_copy(g0, o_hbm.at[pl.ds(cidx * CH, CH)])

        plsc.subcore_barrier()

    @pl.when(cid == 0)
    def _():
        run_block(h0_hbm, o0_hbm)
        run_block(h1_hbm, o1_hbm)

    @pl.when(cid == 1)
    def _():
        run_block(h2_hbm, o2_hbm)
        run_block(h3_hbm, o3_hbm)


def _norm(deg):
    return jnp.where(deg > 0, lax.rsqrt(deg), 0.0)


_R = 1000


def _mm1_body(x_ref, w_ref, dout_ref, o0_ref, o1_ref, o2_ref, o3_ref):
    ns = _norm(dout_ref[...])
    xw = jnp.dot(x_ref[...], w_ref[...],
                 preferred_element_type=jnp.float32) * ns
    o0_ref[...] = xw[:, 0 * H:1 * H]
    o1_ref[...] = xw[:, 1 * H:2 * H]
    o2_ref[...] = xw[:, 2 * H:3 * H]
    o3_ref[...] = xw[:, 3 * H:4 * H]


def _mm2_body(a0_ref, a1_ref, a2_ref, a3_ref, din_ref, dout_ref, w2_ref,
              wlt_ref, blin_ref,
              logits_ref, g0_ref, g1_ref, g2_ref, g3_ref):
    nd = _norm(din_ref[...])
    ns = _norm(dout_ref[...])
    agg = jnp.concatenate(
        [a0_ref[...], a1_ref[...], a2_ref[...], a3_ref[...]], axis=1) * nd
    h1 = jnp.maximum(agg, 0.0)
    logits_ref[...] = jnp.dot(h1, wlt_ref[...],
                              preferred_element_type=jnp.float32) + blin_ref[...]
    g = jnp.dot(h1, w2_ref[...], preferred_element_type=jnp.float32) * ns
    g0_ref[...] = g[:, 0 * H:1 * H]
    g1_ref[...] = g[:, 1 * H:2 * H]
    g2_ref[...] = g[:, 2 * H:3 * H]
    g3_ref[...] = g[:, 3 * H:4 * H]


def _mm3_body(c0_ref, c1_ref, c2_ref, c3_ref, din_ref, x_ref, wst_ref,
              bskip_ref, out_ref, logp_ref):
    nd = _norm(din_ref[...])
    agg = jnp.concatenate(
        [c0_ref[...], c1_ref[...], c2_ref[...], c3_ref[...]], axis=1) * nd
    out = agg + jnp.dot(x_ref[...], wst_ref[...],
                        preferred_element_type=jnp.float32) + bskip_ref[...]
    out_ref[...] = out
    m = jnp.max(out, axis=1, keepdims=True)
    ex = jnp.exp(out - m)
    s = jnp.sum(ex, axis=1, keepdims=True)
    logp_ref[...] = out - m - jnp.log(s)


def _row_spec(cols):
    return pl.BlockSpec((_R, cols), lambda i: (i, 0))


def _full_spec(shape):
    nd = len(shape)
    return pl.BlockSpec(shape, lambda i: (0,) * nd)


def kernel(x, edge_index, W1, b1, W2, b2, Wlin, blin, Wskip, bskip):
    src = edge_index[0].astype(jnp.int32).reshape(NT, NCH, CH)
    dst = edge_index[1].astype(jnp.int32).reshape(NT, NCH, CH)
    ones128 = jnp.ones((128,), jnp.float32)
    zerosN = jnp.zeros((N,), jnp.float32)
    zrow = jnp.zeros((CH, H), jnp.float32)

    deg_out, deg_in = _deg_kernel(src, dst, ones128, zerosN)
    dout_c = deg_out.reshape(N, 1)
    din_c = deg_in.reshape(N, 1)

    grid = N // _R
    hs = pl.pallas_call(
        _mm1_body,
        grid=(grid,),
        in_specs=[_row_spec(D), _full_spec((D, D)), _row_spec(1)],
        out_specs=[_row_spec(H)] * 4,
        out_shape=[jax.ShapeDtypeStruct((N, H), jnp.float32)] * 4,
    )(x, W1, dout_c)

    aggs = _mp_kernel(*hs, src, dst, zrow)

    logits, *gs = pl.pallas_call(
        _mm2_body,
        grid=(grid,),
        in_specs=[_row_spec(H)] * 4 + [_row_spec(1), _row_spec(1),
                  _full_spec((D, D)), _full_spec((D, 64)),
                  _full_spec((1, 64))],
        out_specs=[_row_spec(64)] + [_row_spec(H)] * 4,
        out_shape=[jax.ShapeDtypeStruct((N, 64), jnp.float32)] +
                  [jax.ShapeDtypeStruct((N, H), jnp.float32)] * 4,
    )(*aggs, din_c, dout_c, W2, Wlin.T, blin.reshape(1, 64))

    cs = _mp_kernel(*gs, src, dst, zrow)

    out, logp = pl.pallas_call(
        _mm3_body,
        grid=(grid,),
        in_specs=[_row_spec(H)] * 4 + [_row_spec(1), _row_spec(D),
                  _full_spec((D, D)), _full_spec((1, D))],
        out_specs=[_row_spec(D), _row_spec(D)],
        out_shape=[jax.ShapeDtypeStruct((N, D), jnp.float32)] * 2,
    )(*cs, din_c, x, Wskip.T, bskip.reshape(1, D))

    return (out, logits, logp)
